# baseline (speedup 1.0000x reference)
import jax
import jax.numpy as jnp
from jax import lax
from jax.experimental import pallas as pl
from jax.experimental.pallas import tpu as pltpu
from jax.experimental.pallas import tpu_sc as plsc

N_SPK = 100000
D = 2048
B = 16384
L = 16

NC, NS = 2, 16
NW = NC * NS
B_TC = 2048
B_SC = B - B_TC
B_PER_W = B_SC // NW
CHUNK = 16
N_CHUNKS = B_PER_W // CHUNK
N_SLICES = D // L
U = 8
NBUF = 3

_MAGIC = 0x5F3759DF


def _sc_body(sid_hbm, table_hbm, out_hbm, idx_v, buf0, buf1, buf2,
             gsem0, gsem1, gsem2, ssem0, ssem1, ssem2):
    bufs = (buf0, buf1, buf2)
    gsems = (gsem0, gsem1, gsem2)
    ssems = (ssem0, ssem1, ssem2)
    wid = lax.axis_index("s") * NC + lax.axis_index("c")
    base = wid * B_PER_W
    pltpu.sync_copy(sid_hbm.at[pl.ds(base, B_PER_W)], idx_v)

    def chunk_idx(cc):
        return idx_v[pl.ds(cc * CHUNK, CHUNK)]

    def gather_start(cc, b):
        pltpu.async_copy(table_hbm.at[chunk_idx(cc)], bufs[b], gsems[b])

    def gather_wait(cc, b):
        pltpu.make_async_copy(
            table_hbm.at[chunk_idx(cc)], bufs[b], gsems[b]).wait()

    def store_start(cc, b):
        pltpu.make_async_copy(
            bufs[b], out_hbm.at[pl.ds(base + cc * CHUNK, CHUNK)],
            ssems[b]).start()

    def store_wait(cc, b):
        pltpu.make_async_copy(
            bufs[b], out_hbm.at[pl.ds(base + cc * CHUNK, CHUNK)],
            ssems[b]).wait()

    def pass1(buf):
        lane = lax.iota(jnp.int32, L)
        magic = jnp.full((L,), _MAGIC, jnp.int32)
        svec = jnp.zeros((L,), jnp.float32)
        for r in range(CHUNK):
            def acc_body(j2, accs, r=r):
                j = j2 * U
                return tuple(
                    a + buf[r, pl.ds((j + u) * L, L)] *
                    buf[r, pl.ds((j + u) * L, L)]
                    for u, a in enumerate(accs)
                )

            zeros = tuple(jnp.zeros((L,), jnp.float32) for _ in range(U))
            accs = lax.fori_loop(0, N_SLICES // U, acc_body, zeros)
            a0 = (accs[0] + accs[1]) + (accs[2] + accs[3])
            a1 = (accs[4] + accs[5]) + (accs[6] + accs[7])
            s = a0 + a1
            for sft in (1, 2, 4, 8):
                s = s + s.at[(lane + sft) & (L - 1)].get(
                    mode="promise_in_bounds")
            svec = jnp.where(lane == r, s, svec)

        s_bits = lax.bitcast_convert_type(svec, jnp.int32)
        y = lax.bitcast_convert_type(magic - (s_bits >> 1), jnp.float32)
        half_s = 0.5 * svec
        for _unused in range(3):
            y = y * (1.5 - half_s * y * y)
        return jnp.minimum(y, jnp.float32(1e12))

    def pass2(buf, r_inv_vec):
        lane = lax.iota(jnp.int32, L)
        one = jnp.full((L,), 1.0, jnp.float32)
        for r in range(CHUNK):
            rv = r_inv_vec.at[jnp.full((L,), r, jnp.int32)].get(
                mode="promise_in_bounds")

            def scale_body(j2, _2, r=r, rv=rv):
                j = j2 * U
                for u in range(U):
                    sl = pl.ds((j + u) * L, L)
                    buf[r, sl] = buf[r, sl] * rv
                return 0

            lax.fori_loop(0, N_SLICES // U, scale_body, 0)
            x0 = buf[r, pl.ds(0, L)]
            buf[r, pl.ds(0, L)] = jnp.where(lane == 0, one, x0)

    gather_start(0, 0)
    gather_start(1, 1)

    def group(g, _):
        c = g * NBUF
        for k in range(NBUF):
            cc = c + k

            @pl.when(cc < N_CHUNKS)
            def _do(cc=cc, k=k):
                gather_wait(cc, k)
                r_inv_vec = pass1(bufs[k])

                nb = (k + 2) % NBUF

                @pl.when(cc + 2 < N_CHUNKS)
                def _prefetch():
                    @pl.when(cc >= 1)
                    def _drain():
                        store_wait(cc - 1, nb)

                    gather_start(cc + 2, nb)

                pass2(bufs[k], r_inv_vec)
                store_start(cc, k)

        return 0

    n_groups = (N_CHUNKS + NBUF - 1) // NBUF
    lax.fori_loop(0, n_groups, group, 0)
    store_wait(N_CHUNKS - 2, (N_CHUNKS - 2) % NBUF)
    store_wait(N_CHUNKS - 1, (N_CHUNKS - 1) % NBUF)


def _tc_body(sid_ref, table_ref, o_ref):
    row = table_ref[...]
    s = jnp.sum(row * row)
    inv = 1.0 / jnp.maximum(jnp.sqrt(s), 1e-12)
    out = row * inv
    col = jax.lax.broadcasted_iota(jnp.int32, (1, 1, D), 2)
    o_ref[...] = jnp.where(col == 0, jnp.float32(1.0), out)


@jax.jit
def _reverb_filter_bank(sid, table):
    mesh = plsc.VectorSubcoreMesh(core_axis_name="c", subcore_axis_name="s")
    sc_out = pl.kernel(
        _sc_body,
        out_type=jax.ShapeDtypeStruct((B_SC, D), jnp.float32),
        mesh=mesh,
        scratch_types=[
            pltpu.VMEM((B_PER_W,), jnp.int32),
            pltpu.VMEM((CHUNK, D), jnp.float32),
            pltpu.VMEM((CHUNK, D), jnp.float32),
            pltpu.VMEM((CHUNK, D), jnp.float32),
            pltpu.SemaphoreType.DMA,
            pltpu.SemaphoreType.DMA,
            pltpu.SemaphoreType.DMA,
            pltpu.SemaphoreType.DMA,
            pltpu.SemaphoreType.DMA,
            pltpu.SemaphoreType.DMA,
        ],
    )(sid[:B_SC], table)
    tc_out = pl.pallas_call(
        _tc_body,
        grid_spec=pltpu.PrefetchScalarGridSpec(
            num_scalar_prefetch=1,
            grid=(B_TC,),
            in_specs=[
                pl.BlockSpec((1, 1, D),
                             lambda i, sid_ref: (sid_ref[i], 0, 0)),
            ],
            out_specs=pl.BlockSpec((1, 1, D), lambda i, sid_ref: (i, 0, 0)),
        ),
        out_shape=jax.ShapeDtypeStruct((B_TC, 1, D), jnp.float32),
    )(sid[B_SC:], table[:, None, :])
    return jnp.concatenate([sc_out, tc_out[:, 0, :]], axis=0)


def kernel(sid, table):
    return _reverb_filter_bank(sid.astype(jnp.int32), table)


# speedup vs baseline: 15.6255x; 15.6255x over previous
import jax
import jax.numpy as jnp
from jax import lax
from jax.experimental import pallas as pl
from jax.experimental.pallas import tpu as pltpu
from jax.experimental.pallas import tpu_sc as plsc

N_SPK = 100000
D = 2048
B = 16384
L = 16

NC, NS = 2, 16
NW = NC * NS
B_PER_W = B // NW
CHUNK = 16
N_CHUNKS = B_PER_W // CHUNK
N_SLICES = D // L
U = 8
NBUF = 3

_MAGIC = 0x5F3759DF


def _sc_body(sid_hbm, table_hbm, out_hbm, idx_v, buf0, buf1, buf2,
             gsem0, gsem1, gsem2, ssem0, ssem1, ssem2):
    bufs = (buf0, buf1, buf2)
    gsems = (gsem0, gsem1, gsem2)
    ssems = (ssem0, ssem1, ssem2)
    wid = lax.axis_index("s") * NC + lax.axis_index("c")
    base = wid * B_PER_W
    pltpu.sync_copy(sid_hbm.at[pl.ds(base, B_PER_W)], idx_v)

    def chunk_idx(cc):
        return idx_v[pl.ds(cc * CHUNK, CHUNK)]

    def gather_start(cc, b):
        pltpu.async_copy(table_hbm.at[chunk_idx(cc)], bufs[b], gsems[b])

    def gather_wait(cc, b):
        pltpu.make_async_copy(
            table_hbm.at[chunk_idx(cc)], bufs[b], gsems[b]).wait()

    def store_start(cc, b):
        pltpu.make_async_copy(
            bufs[b], out_hbm.at[pl.ds(base + cc * CHUNK, CHUNK)],
            ssems[b]).start()

    def store_wait(cc, b):
        pltpu.make_async_copy(
            bufs[b], out_hbm.at[pl.ds(base + cc * CHUNK, CHUNK)],
            ssems[b]).wait()

    def pass1(buf):
        lane = lax.iota(jnp.int32, L)
        magic = jnp.full((L,), _MAGIC, jnp.int32)

        def acc_body(j2, accs):
            j = j2 * U
            out = []
            for r in range(CHUNK):
                sq = []
                for u in range(U):
                    x = buf[r, pl.ds((j + u) * L, L)]
                    sq.append(x * x)
                t0 = (sq[0] + sq[1]) + (sq[2] + sq[3])
                t1 = (sq[4] + sq[5]) + (sq[6] + sq[7])
                out.append(accs[r] + (t0 + t1))
            return tuple(out)

        zeros = tuple(jnp.zeros((L,), jnp.float32) for _ in range(CHUNK))
        accs = lax.fori_loop(0, N_SLICES // U, acc_body, zeros)

        vecs = list(accs)
        for k in (1, 2, 4, 8):
            nxt = []
            for i in range(0, len(vecs), 2):
                x, y = vecs[i], vecs[i + 1]
                m = (lane & k) == 0
                sel0 = jnp.where(m, x, y)
                sel1 = jnp.where(m, y, x)
                nxt.append(sel0 + sel1.at[lane ^ k].get(
                    mode="promise_in_bounds"))
            vecs = nxt
        svec = vecs[0]

        s_bits = lax.bitcast_convert_type(svec, jnp.int32)
        y = lax.bitcast_convert_type(magic - (s_bits >> 1), jnp.float32)
        half_s = 0.5 * svec
        for _unused in range(3):
            y = y * (1.5 - half_s * y * y)
        return jnp.minimum(y, jnp.float32(1e12))

    def pass2(buf, r_inv_vec):
        lane = lax.iota(jnp.int32, L)
        one = jnp.full((L,), 1.0, jnp.float32)
        rvs = [
            r_inv_vec.at[jnp.full((L,), r, jnp.int32)].get(
                mode="promise_in_bounds")
            for r in range(CHUNK)
        ]

        def scale_body(j2, _2):
            j = j2 * U
            for r in range(CHUNK):
                for u in range(U):
                    sl = pl.ds((j + u) * L, L)
                    buf[r, sl] = buf[r, sl] * rvs[r]
            return 0

        lax.fori_loop(0, N_SLICES // U, scale_body, 0)
        for r in range(CHUNK):
            x0 = buf[r, pl.ds(0, L)]
            buf[r, pl.ds(0, L)] = jnp.where(lane == 0, one, x0)

    gather_start(0, 0)
    gather_start(1, 1)

    def group(g, _):
        c = g * NBUF
        for k in range(NBUF):
            cc = c + k

            @pl.when(cc < N_CHUNKS)
            def _do(cc=cc, k=k):
                gather_wait(cc, k)
                r_inv_vec = pass1(bufs[k])

                nb = (k + 2) % NBUF

                @pl.when(cc + 2 < N_CHUNKS)
                def _prefetch():
                    @pl.when(cc >= 1)
                    def _drain():
                        store_wait(cc - 1, nb)

                    gather_start(cc + 2, nb)

                pass2(bufs[k], r_inv_vec)
                store_start(cc, k)

        return 0

    n_groups = (N_CHUNKS + NBUF - 1) // NBUF
    lax.fori_loop(0, n_groups, group, 0)
    store_wait(N_CHUNKS - 2, (N_CHUNKS - 2) % NBUF)
    store_wait(N_CHUNKS - 1, (N_CHUNKS - 1) % NBUF)


@jax.jit
def _reverb_filter_bank(sid, table):
    mesh = plsc.VectorSubcoreMesh(core_axis_name="c", subcore_axis_name="s")
    return pl.kernel(
        _sc_body,
        out_type=jax.ShapeDtypeStruct((B, D), jnp.float32),
        mesh=mesh,
        scratch_types=[
            pltpu.VMEM((B_PER_W,), jnp.int32),
            pltpu.VMEM((CHUNK, D), jnp.float32),
            pltpu.VMEM((CHUNK, D), jnp.float32),
            pltpu.VMEM((CHUNK, D), jnp.float32),
            pltpu.SemaphoreType.DMA,
            pltpu.SemaphoreType.DMA,
            pltpu.SemaphoreType.DMA,
            pltpu.SemaphoreType.DMA,
            pltpu.SemaphoreType.DMA,
            pltpu.SemaphoreType.DMA,
        ],
    )(sid, table)


def kernel(sid, table):
    return _reverb_filter_bank(sid.astype(jnp.int32), table)
